# Initial kernel scaffold; baseline (speedup 1.0000x reference)
import jax
import jax.numpy as jnp
from jax.experimental import pallas as pl


def kernel(Ptilde_indices, Ptilde_values, embed, W0, b0, g0, bt0, W1, b1, g1, bt1, W2, b2, g2, bt2):
    raise NotImplementedError("write your pallas kernel here")


# speedup vs baseline: 3.0636x; 3.0636x over previous
import functools

import jax
import jax.numpy as jnp
from jax import lax
from jax.experimental import pallas as pl
from jax.experimental.pallas import tpu as pltpu
from jax.experimental.pallas import tpu_sc as plsc

_N = 50000
_D = 64
_E = 800000
_HALF = 32
_CHUNK = 128
_NSUB = 16
_NCORE = 2
_CHUNKS_TOTAL = _E // _CHUNK
_RPT = 3128
_RPT_LAST = _N - (_NSUB - 1) * _RPT
_RB = 2000
_NB = _N // _RB


_mesh = plsc.VectorSubcoreMesh(core_axis_name="c", subcore_axis_name="s")


@functools.partial(
    pl.kernel,
    out_type=jax.ShapeDtypeStruct((_NCORE, _N, _HALF), jnp.float32),
    mesh=_mesh,
    compiler_params=pltpu.CompilerParams(needs_layout_passes=False,
                                         use_tc_tiling_on_sc=False),
    scratch_types=[
        pltpu.VMEM((_CHUNK,), jnp.int32),
        pltpu.VMEM((_CHUNK,), jnp.int32),
        pltpu.VMEM((_CHUNK,), jnp.int32),
        pltpu.VMEM((_CHUNK,), jnp.float32),
        pltpu.VMEM((_CHUNK, _HALF), jnp.float32),
        pltpu.VMEM_SHARED((_N, _HALF), jnp.float32),
        pltpu.SemaphoreType.DMA,
    ],
)
def _propagate(h2_hbm, col_hbm, row_hbm, val_hbm, zeros_hbm, out_hbm,
               col_v, row_v, idx_v, val_v, gath_v, acc_sh, sem):
    c = lax.axis_index("c")
    s = lax.axis_index("s")

    off = pl.multiple_of(s * _RPT, 8)

    @pl.when(s < _NSUB - 1)
    def _():
        pltpu.sync_copy(zeros_hbm, acc_sh.at[pl.ds(off, _RPT)])

    @pl.when(s == _NSUB - 1)
    def _():
        pltpu.sync_copy(zeros_hbm.at[pl.ds(0, _RPT_LAST)],
                        acc_sh.at[pl.ds(off, _RPT_LAST)])

    plsc.subcore_barrier()

    base_chunks = _CHUNKS_TOTAL // _NSUB
    rem = _CHUNKS_TOTAL % _NSUB
    nchunks = jnp.where(s < rem, base_chunks + 1, base_chunks)

    def body(i, carry):
        k = s + i * _NSUB
        base = k * _CHUNK
        pltpu.sync_copy(col_hbm.at[pl.ds(base, _CHUNK)], col_v)
        pltpu.sync_copy(row_hbm.at[pl.ds(base, _CHUNK)], row_v)
        pltpu.sync_copy(val_hbm.at[pl.ds(base, _CHUNK)], val_v)
        for j in range(_CHUNK // 16):
            cv = col_v[pl.ds(j * 16, 16)]
            idx_v[pl.ds(j * 16, 16)] = cv * 2 + c
        pltpu.async_copy(h2_hbm.at[idx_v], gath_v, sem).wait()
        for g in range(_CHUNK // 16):
            v16 = val_v[pl.ds(g * 16, 16)]
            for l in range(16):
                e = g * 16 + l
                bv = jnp.take_along_axis(
                    v16, jnp.full((16,), l, jnp.int32), axis=0,
                    mode="promise_in_bounds")
                gath_v[e, pl.ds(0, 16)] = gath_v[e, pl.ds(0, 16)] * bv
                gath_v[e, pl.ds(16, 16)] = gath_v[e, pl.ds(16, 16)] * bv
        pltpu.sync_copy(gath_v, acc_sh.at[row_v], add=True)
        return carry

    lax.fori_loop(0, nchunks, body, 0)
    plsc.subcore_barrier()

    @pl.when(s < _NSUB - 1)
    def _():
        pltpu.sync_copy(acc_sh.at[pl.ds(off, _RPT)],
                        out_hbm.at[c, pl.ds(off, _RPT)])

    @pl.when(s == _NSUB - 1)
    def _():
        pltpu.sync_copy(acc_sh.at[pl.ds(off, _RPT_LAST)],
                        out_hbm.at[c, pl.ds(off, _RPT_LAST)])


def _dense0_body(x_ref, w_ref, b_ref, y_ref, stats_ref, acc_ref):
    i = pl.program_id(0)
    y = lax.dot_general(x_ref[...], w_ref[...], (((1,), (1,)), ((), ())),
                        preferred_element_type=jnp.float32) + b_ref[...]
    y_ref[...] = y

    @pl.when(i == 0)
    def _():
        acc_ref[...] = jnp.zeros_like(acc_ref)

    acc_ref[0:1, :] += jnp.sum(y, axis=0, keepdims=True)
    acc_ref[1:2, :] += jnp.sum(y * y, axis=0, keepdims=True)

    @pl.when(i == _NB - 1)
    def _():
        stats_ref[...] = acc_ref[...]


def _dense_halves_body(s_lo_ref, s_hi_ref, w_ref, b_ref, y_ref, stats_ref,
                       acc_ref):
    i = pl.program_id(0)
    w = w_ref[...]
    y = (lax.dot_general(s_lo_ref[0], w[:, :_HALF], (((1,), (1,)), ((), ())),
                         preferred_element_type=jnp.float32)
         + lax.dot_general(s_hi_ref[0], w[:, _HALF:], (((1,), (1,)), ((), ())),
                           preferred_element_type=jnp.float32)
         + b_ref[...])
    y_ref[...] = y

    @pl.when(i == 0)
    def _():
        acc_ref[...] = jnp.zeros_like(acc_ref)

    acc_ref[0:1, :] += jnp.sum(y, axis=0, keepdims=True)
    acc_ref[1:2, :] += jnp.sum(y * y, axis=0, keepdims=True)

    @pl.when(i == _NB - 1)
    def _():
        stats_ref[...] = acc_ref[...]


def _norm_body(y_ref, stats_ref, g_ref, bt_ref, h_ref):
    m = stats_ref[0:1, :] * (1.0 / _N)
    ex2 = stats_ref[1:2, :] * (1.0 / _N)
    v = ex2 - m * m
    inv = g_ref[...] * lax.rsqrt(v + 1e-5)
    h_ref[...] = jnp.maximum(y_ref[...] * inv + (bt_ref[...] - m * inv), 0.0)


def _dense0(x, w, b):
    return pl.pallas_call(
        _dense0_body,
        grid=(_NB,),
        in_specs=[
            pl.BlockSpec((_RB, _D), lambda i: (i, 0)),
            pl.BlockSpec((_D, _D), lambda i: (0, 0)),
            pl.BlockSpec((1, _D), lambda i: (0, 0)),
        ],
        out_specs=[
            pl.BlockSpec((_RB, _D), lambda i: (i, 0)),
            pl.BlockSpec((2, _D), lambda i: (0, 0)),
        ],
        out_shape=[
            jax.ShapeDtypeStruct((_N, _D), jnp.float32),
            jax.ShapeDtypeStruct((2, _D), jnp.float32),
        ],
        scratch_shapes=[pltpu.VMEM((2, _D), jnp.float32)],
    )(x, w, b)


def _dense_halves(s2, w, b):
    return pl.pallas_call(
        _dense_halves_body,
        grid=(_NB,),
        in_specs=[
            pl.BlockSpec((1, _RB, _HALF), lambda i: (0, i, 0)),
            pl.BlockSpec((1, _RB, _HALF), lambda i: (1, i, 0)),
            pl.BlockSpec((_D, _D), lambda i: (0, 0)),
            pl.BlockSpec((1, _D), lambda i: (0, 0)),
        ],
        out_specs=[
            pl.BlockSpec((_RB, _D), lambda i: (i, 0)),
            pl.BlockSpec((2, _D), lambda i: (0, 0)),
        ],
        out_shape=[
            jax.ShapeDtypeStruct((_N, _D), jnp.float32),
            jax.ShapeDtypeStruct((2, _D), jnp.float32),
        ],
        scratch_shapes=[pltpu.VMEM((2, _D), jnp.float32)],
    )(s2, s2, w, b)


def _norm(y, stats, g, bt):
    return pl.pallas_call(
        _norm_body,
        grid=(_NB,),
        in_specs=[
            pl.BlockSpec((_RB, _D), lambda i: (i, 0)),
            pl.BlockSpec((2, _D), lambda i: (0, 0)),
            pl.BlockSpec((1, _D), lambda i: (0, 0)),
            pl.BlockSpec((1, _D), lambda i: (0, 0)),
        ],
        out_specs=pl.BlockSpec((_RB, _D), lambda i: (i, 0)),
        out_shape=jax.ShapeDtypeStruct((_N, _D), jnp.float32),
    )(y, stats, g, bt)


def kernel(Ptilde_indices, Ptilde_values, embed, W0, b0, g0, bt0,
           W1, b1, g1, bt1, W2, b2, g2, bt2):
    row = Ptilde_indices[0]
    col = Ptilde_indices[1]
    zeros = jnp.zeros((_RPT, _HALF), jnp.float32)

    y0, st0 = _dense0(embed, W0, b0.reshape(1, _D))
    H0 = _norm(y0, st0, g0.reshape(1, _D), bt0.reshape(1, _D))

    S1 = _propagate(H0.reshape(2 * _N, _HALF), col, row, Ptilde_values, zeros)
    y1, st1 = _dense_halves(S1, W1, b1.reshape(1, _D))
    H1 = _norm(y1, st1, g1.reshape(1, _D), bt1.reshape(1, _D))

    S2 = _propagate(H1.reshape(2 * _N, _HALF), col, row, Ptilde_values, zeros)
    y2, st2 = _dense_halves(S2, W2, b2.reshape(1, _D))
    H2 = _norm(y2, st2, g2.reshape(1, _D), bt2.reshape(1, _D))

    return (H0, H1, H2)


# speedup vs baseline: 6.5181x; 2.1276x over previous
import functools

import jax
import jax.numpy as jnp
from jax import lax
from jax.experimental import pallas as pl
from jax.experimental.pallas import tpu as pltpu
from jax.experimental.pallas import tpu_sc as plsc

_N = 50000
_D = 64
_E = 800000
_HALF = 32
_CHUNK = 128
_NSUB = 16
_NCORE = 2
_CHUNKS_TOTAL = _E // _CHUNK
_RPT = 3128
_RPT_LAST = _N - (_NSUB - 1) * _RPT
_RB = 2000
_NB = _N // _RB


_mesh = plsc.VectorSubcoreMesh(core_axis_name="c", subcore_axis_name="s")


_BIGC = 8
_BIG = _BIGC * _CHUNK
_NBUF = 4


@functools.partial(
    pl.kernel,
    out_type=jax.ShapeDtypeStruct((_NCORE, _N, _HALF), jnp.float32),
    mesh=_mesh,
    compiler_params=pltpu.CompilerParams(needs_layout_passes=False,
                                         use_tc_tiling_on_sc=False),
    scratch_types=[
        pltpu.VMEM((_BIG,), jnp.int32),
        pltpu.VMEM((_BIG,), jnp.int32),
        pltpu.VMEM((_BIG,), jnp.float32),
        [pltpu.VMEM((_CHUNK,), jnp.int32) for _ in range(_NBUF)],
        [pltpu.VMEM((_CHUNK,), jnp.int32) for _ in range(_NBUF)],
        [pltpu.VMEM((_CHUNK, _HALF), jnp.float32) for _ in range(_NBUF)],
        [pltpu.SemaphoreType.DMA for _ in range(_NBUF)],
        [pltpu.SemaphoreType.DMA for _ in range(_NBUF)],
        pltpu.VMEM_SHARED((_N, _HALF), jnp.float32),
    ],
)
def _propagate(h2_hbm, col_hbm, row_hbm, val_hbm, zeros_hbm, out_hbm,
               colb, rowb, valb, idx_bufs, row_bufs, gath_bufs, gsems, ssems,
               acc_sh):
    c = lax.axis_index("c")
    s = lax.axis_index("s")

    off = pl.multiple_of(s * _RPT, 8)

    @pl.when(s < _NSUB - 1)
    def _():
        pltpu.sync_copy(zeros_hbm, acc_sh.at[pl.ds(off, _RPT)])

    @pl.when(s == _NSUB - 1)
    def _():
        pltpu.sync_copy(zeros_hbm.at[pl.ds(0, _RPT_LAST)],
                        acc_sh.at[pl.ds(off, _RPT_LAST)])

    plsc.subcore_barrier()

    base_chunks = _CHUNKS_TOTAL // _NSUB
    rem = _CHUNKS_TOTAL % _NSUB
    nchunks = base_chunks + jnp.where(s < rem, 1, 0)
    start_chunk = s * base_chunks + jnp.minimum(s, rem)
    nbig = nchunks // _BIGC

    def scale(j):
        buf = gath_bufs[j % _NBUF]
        for g in range(_CHUNK // 16):
            v16 = valb[pl.ds(j * _CHUNK + g * 16, 16)]
            for l in range(16):
                e = g * 16 + l
                bv = jnp.take_along_axis(
                    v16, jnp.full((16,), l, jnp.int32), axis=0,
                    mode="promise_in_bounds")
                buf[e, pl.ds(0, 16)] = buf[e, pl.ds(0, 16)] * bv
                buf[e, pl.ds(16, 16)] = buf[e, pl.ds(16, 16)] * bv

    def big_body(i, carry):
        ebase = pl.multiple_of((start_chunk + i * _BIGC) * _CHUNK, 8)
        pltpu.sync_copy(col_hbm.at[pl.ds(ebase, _BIG)], colb)
        pltpu.sync_copy(row_hbm.at[pl.ds(ebase, _BIG)], rowb)
        pltpu.sync_copy(val_hbm.at[pl.ds(ebase, _BIG)], valb)

        gh = [None] * _BIGC
        sh = [None] * _BIGC

        def prep_start(j):
            b = j % _NBUF
            for g in range(_CHUNK // 16):
                cv = colb[pl.ds(j * _CHUNK + g * 16, 16)]
                idx_bufs[b][pl.ds(g * 16, 16)] = cv * 2 + c
                row_bufs[b][pl.ds(g * 16, 16)] = (
                    rowb[pl.ds(j * _CHUNK + g * 16, 16)])
            gh[j] = pltpu.async_copy(h2_hbm.at[idx_bufs[b]], gath_bufs[b],
                                     gsems[b])

        for j in range(_NBUF - 1):
            prep_start(j)
        for j in range(_BIGC):
            nj = j + _NBUF - 1
            if nj < _BIGC:
                pj = nj - _NBUF
                if pj >= 0:
                    sh[pj].wait()
                prep_start(nj)
            gh[j].wait()
            scale(j)
            sh[j] = pltpu.async_copy(gath_bufs[j % _NBUF],
                                     acc_sh.at[row_bufs[j % _NBUF]],
                                     ssems[j % _NBUF], add=True)
        for j in range(max(0, _BIGC - _NBUF), _BIGC):
            sh[j].wait()
        return carry

    lax.fori_loop(0, nbig, big_body, 0)

    def tail_body(t, carry):
        base = pl.multiple_of((start_chunk + nbig * _BIGC + t) * _CHUNK, 8)
        pltpu.sync_copy(col_hbm.at[pl.ds(base, _CHUNK)], idx_bufs[1])
        pltpu.sync_copy(row_hbm.at[pl.ds(base, _CHUNK)], row_bufs[0])
        pltpu.sync_copy(val_hbm.at[pl.ds(base, _CHUNK)],
                        valb.at[pl.ds(0, _CHUNK)])
        for g in range(_CHUNK // 16):
            cv = idx_bufs[1][pl.ds(g * 16, 16)]
            idx_bufs[0][pl.ds(g * 16, 16)] = cv * 2 + c
        pltpu.async_copy(h2_hbm.at[idx_bufs[0]], gath_bufs[0],
                         gsems[0]).wait()
        scale(0)
        pltpu.sync_copy(gath_bufs[0], acc_sh.at[row_bufs[0]], add=True)
        return carry

    lax.fori_loop(0, nchunks - nbig * _BIGC, tail_body, 0)
    plsc.subcore_barrier()

    @pl.when(s < _NSUB - 1)
    def _():
        pltpu.sync_copy(acc_sh.at[pl.ds(off, _RPT)],
                        out_hbm.at[c, pl.ds(off, _RPT)])

    @pl.when(s == _NSUB - 1)
    def _():
        pltpu.sync_copy(acc_sh.at[pl.ds(off, _RPT_LAST)],
                        out_hbm.at[c, pl.ds(off, _RPT_LAST)])


def _dense0_body(x_ref, w_ref, b_ref, y_ref, stats_ref, acc_ref):
    i = pl.program_id(0)
    y = lax.dot_general(x_ref[...], w_ref[...], (((1,), (1,)), ((), ())),
                        preferred_element_type=jnp.float32) + b_ref[...]
    y_ref[...] = y

    @pl.when(i == 0)
    def _():
        acc_ref[...] = jnp.zeros_like(acc_ref)

    acc_ref[0:1, :] += jnp.sum(y, axis=0, keepdims=True)
    acc_ref[1:2, :] += jnp.sum(y * y, axis=0, keepdims=True)

    @pl.when(i == _NB - 1)
    def _():
        stats_ref[...] = acc_ref[...]


def _dense_halves_body(s_lo_ref, s_hi_ref, w_ref, b_ref, y_ref, stats_ref,
                       acc_ref):
    i = pl.program_id(0)
    w = w_ref[...]
    y = (lax.dot_general(s_lo_ref[0], w[:, :_HALF], (((1,), (1,)), ((), ())),
                         preferred_element_type=jnp.float32)
         + lax.dot_general(s_hi_ref[0], w[:, _HALF:], (((1,), (1,)), ((), ())),
                           preferred_element_type=jnp.float32)
         + b_ref[...])
    y_ref[...] = y

    @pl.when(i == 0)
    def _():
        acc_ref[...] = jnp.zeros_like(acc_ref)

    acc_ref[0:1, :] += jnp.sum(y, axis=0, keepdims=True)
    acc_ref[1:2, :] += jnp.sum(y * y, axis=0, keepdims=True)

    @pl.when(i == _NB - 1)
    def _():
        stats_ref[...] = acc_ref[...]


def _norm_body(y_ref, stats_ref, g_ref, bt_ref, h_ref):
    m = stats_ref[0:1, :] * (1.0 / _N)
    ex2 = stats_ref[1:2, :] * (1.0 / _N)
    v = ex2 - m * m
    inv = g_ref[...] * lax.rsqrt(v + 1e-5)
    h_ref[...] = jnp.maximum(y_ref[...] * inv + (bt_ref[...] - m * inv), 0.0)


def _dense0(x, w, b):
    return pl.pallas_call(
        _dense0_body,
        grid=(_NB,),
        in_specs=[
            pl.BlockSpec((_RB, _D), lambda i: (i, 0)),
            pl.BlockSpec((_D, _D), lambda i: (0, 0)),
            pl.BlockSpec((1, _D), lambda i: (0, 0)),
        ],
        out_specs=[
            pl.BlockSpec((_RB, _D), lambda i: (i, 0)),
            pl.BlockSpec((2, _D), lambda i: (0, 0)),
        ],
        out_shape=[
            jax.ShapeDtypeStruct((_N, _D), jnp.float32),
            jax.ShapeDtypeStruct((2, _D), jnp.float32),
        ],
        scratch_shapes=[pltpu.VMEM((2, _D), jnp.float32)],
    )(x, w, b)


def _dense_halves(s2, w, b):
    return pl.pallas_call(
        _dense_halves_body,
        grid=(_NB,),
        in_specs=[
            pl.BlockSpec((1, _RB, _HALF), lambda i: (0, i, 0)),
            pl.BlockSpec((1, _RB, _HALF), lambda i: (1, i, 0)),
            pl.BlockSpec((_D, _D), lambda i: (0, 0)),
            pl.BlockSpec((1, _D), lambda i: (0, 0)),
        ],
        out_specs=[
            pl.BlockSpec((_RB, _D), lambda i: (i, 0)),
            pl.BlockSpec((2, _D), lambda i: (0, 0)),
        ],
        out_shape=[
            jax.ShapeDtypeStruct((_N, _D), jnp.float32),
            jax.ShapeDtypeStruct((2, _D), jnp.float32),
        ],
        scratch_shapes=[pltpu.VMEM((2, _D), jnp.float32)],
    )(s2, s2, w, b)


def _norm(y, stats, g, bt):
    return pl.pallas_call(
        _norm_body,
        grid=(_NB,),
        in_specs=[
            pl.BlockSpec((_RB, _D), lambda i: (i, 0)),
            pl.BlockSpec((2, _D), lambda i: (0, 0)),
            pl.BlockSpec((1, _D), lambda i: (0, 0)),
            pl.BlockSpec((1, _D), lambda i: (0, 0)),
        ],
        out_specs=pl.BlockSpec((_RB, _D), lambda i: (i, 0)),
        out_shape=jax.ShapeDtypeStruct((_N, _D), jnp.float32),
    )(y, stats, g, bt)


def kernel(Ptilde_indices, Ptilde_values, embed, W0, b0, g0, bt0,
           W1, b1, g1, bt1, W2, b2, g2, bt2):
    row = Ptilde_indices[0]
    col = Ptilde_indices[1]
    zeros = jnp.zeros((_RPT, _HALF), jnp.float32)

    y0, st0 = _dense0(embed, W0, b0.reshape(1, _D))
    H0 = _norm(y0, st0, g0.reshape(1, _D), bt0.reshape(1, _D))

    S1 = _propagate(H0.reshape(2 * _N, _HALF), col, row, Ptilde_values, zeros)
    y1, st1 = _dense_halves(S1, W1, b1.reshape(1, _D))
    H1 = _norm(y1, st1, g1.reshape(1, _D), bt1.reshape(1, _D))

    S2 = _propagate(H1.reshape(2 * _N, _HALF), col, row, Ptilde_values, zeros)
    y2, st2 = _dense_halves(S2, W2, b2.reshape(1, _D))
    H2 = _norm(y2, st2, g2.reshape(1, _D), bt2.reshape(1, _D))

    return (H0, H1, H2)


# speedup vs baseline: 8.4046x; 1.2894x over previous
import functools

import jax
import jax.numpy as jnp
from jax import lax
from jax.experimental import pallas as pl
from jax.experimental.pallas import tpu as pltpu
from jax.experimental.pallas import tpu_sc as plsc

_N = 50000
_D = 64
_E = 800000
_HALF = 32
_CHUNK = 128
_NSUB = 16
_NCORE = 2
_CHUNKS_TOTAL = _E // _CHUNK
_RPT = 3128
_RPT_LAST = _N - (_NSUB - 1) * _RPT
_RB = 2000
_NB = _N // _RB


_mesh = plsc.VectorSubcoreMesh(core_axis_name="c", subcore_axis_name="s")


_BIGC = 16
_BIG = _BIGC * _CHUNK
_NBUF = 4


@functools.partial(
    pl.kernel,
    out_type=jax.ShapeDtypeStruct((_NCORE, _N, _HALF), jnp.float32),
    mesh=_mesh,
    compiler_params=pltpu.CompilerParams(needs_layout_passes=False,
                                         use_tc_tiling_on_sc=False),
    scratch_types=[
        pltpu.VMEM((_BIG,), jnp.int32),
        pltpu.VMEM((_BIG,), jnp.int32),
        pltpu.VMEM((_BIG,), jnp.float32),
        [pltpu.VMEM((_CHUNK,), jnp.int32) for _ in range(_NBUF)],
        [pltpu.VMEM((_CHUNK,), jnp.int32) for _ in range(_NBUF)],
        [pltpu.VMEM((_CHUNK, _HALF), jnp.float32) for _ in range(_NBUF)],
        [pltpu.SemaphoreType.DMA for _ in range(_NBUF)],
        [pltpu.SemaphoreType.DMA for _ in range(_NBUF)],
        [pltpu.SemaphoreType.DMA for _ in range(3)],
        pltpu.VMEM_SHARED((_N, _HALF), jnp.float32),
    ],
)
def _propagate(h2_hbm, col_hbm, row_hbm, val_hbm, zeros_hbm, out_hbm,
               colb, rowb, valb, idx_bufs, row_bufs, gath_bufs, gsems, ssems,
               isems, acc_sh):
    c = lax.axis_index("c")
    s = lax.axis_index("s")

    off = pl.multiple_of(s * _RPT, 8)

    @pl.when(s < _NSUB - 1)
    def _():
        pltpu.sync_copy(zeros_hbm, acc_sh.at[pl.ds(off, _RPT)])

    @pl.when(s == _NSUB - 1)
    def _():
        pltpu.sync_copy(zeros_hbm.at[pl.ds(0, _RPT_LAST)],
                        acc_sh.at[pl.ds(off, _RPT_LAST)])

    plsc.subcore_barrier()

    base_chunks = _CHUNKS_TOTAL // _NSUB
    rem = _CHUNKS_TOTAL % _NSUB
    nchunks = base_chunks + jnp.where(s < rem, 1, 0)
    start_chunk = s * base_chunks + jnp.minimum(s, rem)
    nbig = nchunks // _BIGC

    def scale(j):
        buf = gath_bufs[j % _NBUF]

        def sbody(g, carry):
            voff = pl.multiple_of(j * _CHUNK + g * 16, 16)
            v16 = valb[pl.ds(voff, 16)]
            e0 = g * 16
            for l in range(16):
                e = e0 + l
                bv = jnp.take_along_axis(
                    v16, jnp.full((16,), l, jnp.int32), axis=0,
                    mode="promise_in_bounds")
                buf[e, pl.ds(0, 16)] = buf[e, pl.ds(0, 16)] * bv
                buf[e, pl.ds(16, 16)] = buf[e, pl.ds(16, 16)] * bv
            return carry

        lax.fori_loop(0, _CHUNK // 16, sbody, 0)

    def big_body(i, carry):
        ebase = pl.multiple_of((start_chunk + i * _BIGC) * _CHUNK, 8)
        hc = pltpu.async_copy(col_hbm.at[pl.ds(ebase, _BIG)], colb, isems[0])
        hr = pltpu.async_copy(row_hbm.at[pl.ds(ebase, _BIG)], rowb, isems[1])
        hv = pltpu.async_copy(val_hbm.at[pl.ds(ebase, _BIG)], valb, isems[2])
        hc.wait()
        hr.wait()
        hv.wait()

        gh = [None] * _BIGC
        sh = [None] * _BIGC

        def prep_start(j):
            b = j % _NBUF
            for g in range(_CHUNK // 16):
                cv = colb[pl.ds(j * _CHUNK + g * 16, 16)]
                idx_bufs[b][pl.ds(g * 16, 16)] = cv * 2 + c
                row_bufs[b][pl.ds(g * 16, 16)] = (
                    rowb[pl.ds(j * _CHUNK + g * 16, 16)])
            gh[j] = pltpu.async_copy(h2_hbm.at[idx_bufs[b]], gath_bufs[b],
                                     gsems[b])

        for j in range(_NBUF - 1):
            prep_start(j)
        for j in range(_BIGC):
            nj = j + _NBUF - 1
            if nj < _BIGC:
                pj = nj - _NBUF
                if pj >= 0:
                    sh[pj].wait()
                prep_start(nj)
            gh[j].wait()
            scale(j)
            sh[j] = pltpu.async_copy(gath_bufs[j % _NBUF],
                                     acc_sh.at[row_bufs[j % _NBUF]],
                                     ssems[j % _NBUF], add=True)
        for j in range(max(0, _BIGC - _NBUF), _BIGC):
            sh[j].wait()
        return carry

    lax.fori_loop(0, nbig, big_body, 0)

    def tail_body(t, carry):
        base = pl.multiple_of((start_chunk + nbig * _BIGC + t) * _CHUNK, 8)
        pltpu.sync_copy(col_hbm.at[pl.ds(base, _CHUNK)], idx_bufs[1])
        pltpu.sync_copy(row_hbm.at[pl.ds(base, _CHUNK)], row_bufs[0])
        pltpu.sync_copy(val_hbm.at[pl.ds(base, _CHUNK)],
                        valb.at[pl.ds(0, _CHUNK)])
        for g in range(_CHUNK // 16):
            cv = idx_bufs[1][pl.ds(g * 16, 16)]
            idx_bufs[0][pl.ds(g * 16, 16)] = cv * 2 + c
        pltpu.async_copy(h2_hbm.at[idx_bufs[0]], gath_bufs[0],
                         gsems[0]).wait()
        scale(0)
        pltpu.sync_copy(gath_bufs[0], acc_sh.at[row_bufs[0]], add=True)
        return carry

    lax.fori_loop(0, nchunks - nbig * _BIGC, tail_body, 0)
    plsc.subcore_barrier()

    @pl.when(s < _NSUB - 1)
    def _():
        pltpu.sync_copy(acc_sh.at[pl.ds(off, _RPT)],
                        out_hbm.at[c, pl.ds(off, _RPT)])

    @pl.when(s == _NSUB - 1)
    def _():
        pltpu.sync_copy(acc_sh.at[pl.ds(off, _RPT_LAST)],
                        out_hbm.at[c, pl.ds(off, _RPT_LAST)])


def _dense0_body(x_ref, w_ref, b_ref, y_ref, stats_ref, acc_ref):
    i = pl.program_id(0)
    y = lax.dot_general(x_ref[...], w_ref[...], (((1,), (1,)), ((), ())),
                        preferred_element_type=jnp.float32) + b_ref[...]
    y_ref[...] = y

    @pl.when(i == 0)
    def _():
        acc_ref[...] = jnp.zeros_like(acc_ref)

    acc_ref[0:1, :] += jnp.sum(y, axis=0, keepdims=True)
    acc_ref[1:2, :] += jnp.sum(y * y, axis=0, keepdims=True)

    @pl.when(i == _NB - 1)
    def _():
        stats_ref[...] = acc_ref[...]


def _dense_halves_body(s_lo_ref, s_hi_ref, w_ref, b_ref, y_ref, stats_ref,
                       acc_ref):
    i = pl.program_id(0)
    w = w_ref[...]
    y = (lax.dot_general(s_lo_ref[0], w[:, :_HALF], (((1,), (1,)), ((), ())),
                         preferred_element_type=jnp.float32)
         + lax.dot_general(s_hi_ref[0], w[:, _HALF:], (((1,), (1,)), ((), ())),
                           preferred_element_type=jnp.float32)
         + b_ref[...])
    y_ref[...] = y

    @pl.when(i == 0)
    def _():
        acc_ref[...] = jnp.zeros_like(acc_ref)

    acc_ref[0:1, :] += jnp.sum(y, axis=0, keepdims=True)
    acc_ref[1:2, :] += jnp.sum(y * y, axis=0, keepdims=True)

    @pl.when(i == _NB - 1)
    def _():
        stats_ref[...] = acc_ref[...]


def _norm_body(y_ref, stats_ref, g_ref, bt_ref, h_ref):
    m = stats_ref[0:1, :] * (1.0 / _N)
    ex2 = stats_ref[1:2, :] * (1.0 / _N)
    v = ex2 - m * m
    inv = g_ref[...] * lax.rsqrt(v + 1e-5)
    h_ref[...] = jnp.maximum(y_ref[...] * inv + (bt_ref[...] - m * inv), 0.0)


def _dense0(x, w, b):
    return pl.pallas_call(
        _dense0_body,
        grid=(_NB,),
        in_specs=[
            pl.BlockSpec((_RB, _D), lambda i: (i, 0)),
            pl.BlockSpec((_D, _D), lambda i: (0, 0)),
            pl.BlockSpec((1, _D), lambda i: (0, 0)),
        ],
        out_specs=[
            pl.BlockSpec((_RB, _D), lambda i: (i, 0)),
            pl.BlockSpec((2, _D), lambda i: (0, 0)),
        ],
        out_shape=[
            jax.ShapeDtypeStruct((_N, _D), jnp.float32),
            jax.ShapeDtypeStruct((2, _D), jnp.float32),
        ],
        scratch_shapes=[pltpu.VMEM((2, _D), jnp.float32)],
    )(x, w, b)


def _dense_halves(s2, w, b):
    return pl.pallas_call(
        _dense_halves_body,
        grid=(_NB,),
        in_specs=[
            pl.BlockSpec((1, _RB, _HALF), lambda i: (0, i, 0)),
            pl.BlockSpec((1, _RB, _HALF), lambda i: (1, i, 0)),
            pl.BlockSpec((_D, _D), lambda i: (0, 0)),
            pl.BlockSpec((1, _D), lambda i: (0, 0)),
        ],
        out_specs=[
            pl.BlockSpec((_RB, _D), lambda i: (i, 0)),
            pl.BlockSpec((2, _D), lambda i: (0, 0)),
        ],
        out_shape=[
            jax.ShapeDtypeStruct((_N, _D), jnp.float32),
            jax.ShapeDtypeStruct((2, _D), jnp.float32),
        ],
        scratch_shapes=[pltpu.VMEM((2, _D), jnp.float32)],
    )(s2, s2, w, b)


def _norm(y, stats, g, bt):
    return pl.pallas_call(
        _norm_body,
        grid=(_NB,),
        in_specs=[
            pl.BlockSpec((_RB, _D), lambda i: (i, 0)),
            pl.BlockSpec((2, _D), lambda i: (0, 0)),
            pl.BlockSpec((1, _D), lambda i: (0, 0)),
            pl.BlockSpec((1, _D), lambda i: (0, 0)),
        ],
        out_specs=pl.BlockSpec((_RB, _D), lambda i: (i, 0)),
        out_shape=jax.ShapeDtypeStruct((_N, _D), jnp.float32),
    )(y, stats, g, bt)


def kernel(Ptilde_indices, Ptilde_values, embed, W0, b0, g0, bt0,
           W1, b1, g1, bt1, W2, b2, g2, bt2):
    row = Ptilde_indices[0]
    col = Ptilde_indices[1]
    zeros = jnp.zeros((_RPT, _HALF), jnp.float32)

    y0, st0 = _dense0(embed, W0, b0.reshape(1, _D))
    H0 = _norm(y0, st0, g0.reshape(1, _D), bt0.reshape(1, _D))

    S1 = _propagate(H0.reshape(2 * _N, _HALF), col, row, Ptilde_values, zeros)
    y1, st1 = _dense_halves(S1, W1, b1.reshape(1, _D))
    H1 = _norm(y1, st1, g1.reshape(1, _D), bt1.reshape(1, _D))

    S2 = _propagate(H1.reshape(2 * _N, _HALF), col, row, Ptilde_values, zeros)
    y2, st2 = _dense_halves(S2, W2, b2.reshape(1, _D))
    H2 = _norm(y2, st2, g2.reshape(1, _D), bt2.reshape(1, _D))

    return (H0, H1, H2)


# speedup vs baseline: 8.5558x; 1.0180x over previous
import functools

import jax
import jax.numpy as jnp
from jax import lax
from jax.experimental import pallas as pl
from jax.experimental.pallas import tpu as pltpu
from jax.experimental.pallas import tpu_sc as plsc

_N = 50000
_D = 64
_E = 800000
_HALF = 32
_CHUNK = 128
_NSUB = 16
_NCORE = 2
_EPAD = 819200
_CHUNKS_TOTAL = _EPAD // _CHUNK
_TILE_CHUNKS = _CHUNKS_TOTAL // _NSUB
_RPT = 3128
_RPT_LAST = _N - (_NSUB - 1) * _RPT
_RB = 2000
_NB = _N // _RB


_mesh = plsc.VectorSubcoreMesh(core_axis_name="c", subcore_axis_name="s")


_BIGC = 16
_BIG = _BIGC * _CHUNK
_NBUF = 4


@functools.partial(
    pl.kernel,
    out_type=jax.ShapeDtypeStruct((_NCORE, _N, _HALF), jnp.float32),
    mesh=_mesh,
    compiler_params=pltpu.CompilerParams(needs_layout_passes=False,
                                         use_tc_tiling_on_sc=False),
    scratch_types=[
        pltpu.VMEM((_BIG,), jnp.int32),
        pltpu.VMEM((_BIG,), jnp.int32),
        pltpu.VMEM((_BIG,), jnp.float32),
        [pltpu.VMEM((_CHUNK,), jnp.int32) for _ in range(_NBUF)],
        [pltpu.VMEM((_CHUNK,), jnp.int32) for _ in range(_NBUF)],
        [pltpu.VMEM((_CHUNK, _HALF), jnp.float32) for _ in range(_NBUF)],
        [pltpu.SemaphoreType.DMA for _ in range(_NBUF)],
        [pltpu.SemaphoreType.DMA for _ in range(_NBUF)],
        [pltpu.SemaphoreType.DMA for _ in range(3)],
        pltpu.VMEM_SHARED((_N, _HALF), jnp.float32),
    ],
)
def _propagate(h2_hbm, col_hbm, row_hbm, val_hbm, zeros_hbm, out_hbm,
               colb, rowb, valb, idx_bufs, row_bufs, gath_bufs, gsems, ssems,
               isems, acc_sh):
    c = lax.axis_index("c")
    s = lax.axis_index("s")

    off = pl.multiple_of(s * _RPT, 8)

    @pl.when(s < _NSUB - 1)
    def _():
        pltpu.sync_copy(zeros_hbm, acc_sh.at[pl.ds(off, _RPT)])

    @pl.when(s == _NSUB - 1)
    def _():
        pltpu.sync_copy(zeros_hbm.at[pl.ds(0, _RPT_LAST)],
                        acc_sh.at[pl.ds(off, _RPT_LAST)])

    plsc.subcore_barrier()

    start_chunk = s * _TILE_CHUNKS
    nbig = _TILE_CHUNKS // _BIGC

    def scale(j):
        buf = gath_bufs[j % _NBUF]

        def sbody(g, carry):
            voff = pl.multiple_of(j * _CHUNK + g * 16, 16)
            v16 = valb[pl.ds(voff, 16)]
            e0 = g * 16
            for l in range(16):
                e = e0 + l
                bv = jnp.take_along_axis(
                    v16, jnp.full((16,), l, jnp.int32), axis=0,
                    mode="promise_in_bounds")
                buf[e, pl.ds(0, 16)] = buf[e, pl.ds(0, 16)] * bv
                buf[e, pl.ds(16, 16)] = buf[e, pl.ds(16, 16)] * bv
            return carry

        lax.fori_loop(0, _CHUNK // 16, sbody, 0)

    def big_body(i, carry):
        ebase = pl.multiple_of((start_chunk + i * _BIGC) * _CHUNK, 8)
        hc = pltpu.async_copy(col_hbm.at[pl.ds(ebase, _BIG)], colb, isems[0])
        hr = pltpu.async_copy(row_hbm.at[pl.ds(ebase, _BIG)], rowb, isems[1])
        hv = pltpu.async_copy(val_hbm.at[pl.ds(ebase, _BIG)], valb, isems[2])
        hc.wait()
        hr.wait()
        hv.wait()

        gh = [None] * _BIGC
        sh = [None] * _BIGC

        def prep_start(j):
            b = j % _NBUF
            for g in range(_CHUNK // 16):
                cv = colb[pl.ds(j * _CHUNK + g * 16, 16)]
                idx_bufs[b][pl.ds(g * 16, 16)] = cv * 2 + c
                row_bufs[b][pl.ds(g * 16, 16)] = (
                    rowb[pl.ds(j * _CHUNK + g * 16, 16)])
            gh[j] = pltpu.async_copy(h2_hbm.at[idx_bufs[b]], gath_bufs[b],
                                     gsems[b])

        for j in range(_NBUF - 1):
            prep_start(j)
        for j in range(_BIGC):
            nj = j + _NBUF - 1
            if nj < _BIGC:
                pj = nj - _NBUF
                if pj >= 0:
                    sh[pj].wait()
                prep_start(nj)
            gh[j].wait()
            scale(j)
            sh[j] = pltpu.async_copy(gath_bufs[j % _NBUF],
                                     acc_sh.at[row_bufs[j % _NBUF]],
                                     ssems[j % _NBUF], add=True)
        for j in range(max(0, _BIGC - _NBUF), _BIGC):
            sh[j].wait()
        return carry

    lax.fori_loop(0, nbig, big_body, 0)
    plsc.subcore_barrier()

    @pl.when(s < _NSUB - 1)
    def _():
        pltpu.sync_copy(acc_sh.at[pl.ds(off, _RPT)],
                        out_hbm.at[c, pl.ds(off, _RPT)])

    @pl.when(s == _NSUB - 1)
    def _():
        pltpu.sync_copy(acc_sh.at[pl.ds(off, _RPT_LAST)],
                        out_hbm.at[c, pl.ds(off, _RPT_LAST)])


def _dense0_body(x_ref, w_ref, b_ref, y_ref, stats_ref, acc_ref):
    i = pl.program_id(0)
    y = lax.dot_general(x_ref[...], w_ref[...], (((1,), (1,)), ((), ())),
                        preferred_element_type=jnp.float32) + b_ref[...]
    y_ref[...] = y

    @pl.when(i == 0)
    def _():
        acc_ref[...] = jnp.zeros_like(acc_ref)

    acc_ref[0:1, :] += jnp.sum(y, axis=0, keepdims=True)
    acc_ref[1:2, :] += jnp.sum(y * y, axis=0, keepdims=True)

    @pl.when(i == _NB - 1)
    def _():
        stats_ref[...] = acc_ref[...]


def _dense_halves_body(s_lo_ref, s_hi_ref, w_ref, b_ref, y_ref, stats_ref,
                       acc_ref):
    i = pl.program_id(0)
    w = w_ref[...]
    y = (lax.dot_general(s_lo_ref[0], w[:, :_HALF], (((1,), (1,)), ((), ())),
                         preferred_element_type=jnp.float32)
         + lax.dot_general(s_hi_ref[0], w[:, _HALF:], (((1,), (1,)), ((), ())),
                           preferred_element_type=jnp.float32)
         + b_ref[...])
    y_ref[...] = y

    @pl.when(i == 0)
    def _():
        acc_ref[...] = jnp.zeros_like(acc_ref)

    acc_ref[0:1, :] += jnp.sum(y, axis=0, keepdims=True)
    acc_ref[1:2, :] += jnp.sum(y * y, axis=0, keepdims=True)

    @pl.when(i == _NB - 1)
    def _():
        stats_ref[...] = acc_ref[...]


def _norm_body(y_ref, stats_ref, g_ref, bt_ref, h_ref):
    m = stats_ref[0:1, :] * (1.0 / _N)
    ex2 = stats_ref[1:2, :] * (1.0 / _N)
    v = ex2 - m * m
    inv = g_ref[...] * lax.rsqrt(v + 1e-5)
    h_ref[...] = jnp.maximum(y_ref[...] * inv + (bt_ref[...] - m * inv), 0.0)


def _dense0(x, w, b):
    return pl.pallas_call(
        _dense0_body,
        grid=(_NB,),
        in_specs=[
            pl.BlockSpec((_RB, _D), lambda i: (i, 0)),
            pl.BlockSpec((_D, _D), lambda i: (0, 0)),
            pl.BlockSpec((1, _D), lambda i: (0, 0)),
        ],
        out_specs=[
            pl.BlockSpec((_RB, _D), lambda i: (i, 0)),
            pl.BlockSpec((2, _D), lambda i: (0, 0)),
        ],
        out_shape=[
            jax.ShapeDtypeStruct((_N, _D), jnp.float32),
            jax.ShapeDtypeStruct((2, _D), jnp.float32),
        ],
        scratch_shapes=[pltpu.VMEM((2, _D), jnp.float32)],
    )(x, w, b)


def _dense_halves(s2, w, b):
    return pl.pallas_call(
        _dense_halves_body,
        grid=(_NB,),
        in_specs=[
            pl.BlockSpec((1, _RB, _HALF), lambda i: (0, i, 0)),
            pl.BlockSpec((1, _RB, _HALF), lambda i: (1, i, 0)),
            pl.BlockSpec((_D, _D), lambda i: (0, 0)),
            pl.BlockSpec((1, _D), lambda i: (0, 0)),
        ],
        out_specs=[
            pl.BlockSpec((_RB, _D), lambda i: (i, 0)),
            pl.BlockSpec((2, _D), lambda i: (0, 0)),
        ],
        out_shape=[
            jax.ShapeDtypeStruct((_N, _D), jnp.float32),
            jax.ShapeDtypeStruct((2, _D), jnp.float32),
        ],
        scratch_shapes=[pltpu.VMEM((2, _D), jnp.float32)],
    )(s2, s2, w, b)


def _norm(y, stats, g, bt):
    return pl.pallas_call(
        _norm_body,
        grid=(_NB,),
        in_specs=[
            pl.BlockSpec((_RB, _D), lambda i: (i, 0)),
            pl.BlockSpec((2, _D), lambda i: (0, 0)),
            pl.BlockSpec((1, _D), lambda i: (0, 0)),
            pl.BlockSpec((1, _D), lambda i: (0, 0)),
        ],
        out_specs=pl.BlockSpec((_RB, _D), lambda i: (i, 0)),
        out_shape=jax.ShapeDtypeStruct((_N, _D), jnp.float32),
    )(y, stats, g, bt)


def kernel(Ptilde_indices, Ptilde_values, embed, W0, b0, g0, bt0,
           W1, b1, g1, bt1, W2, b2, g2, bt2):
    pad_idx = (jnp.arange(_EPAD - _E, dtype=jnp.int32) * 131) % _N
    row = jnp.concatenate([Ptilde_indices[0], pad_idx])
    col = jnp.concatenate([Ptilde_indices[1], pad_idx])
    val = jnp.concatenate(
        [Ptilde_values, jnp.zeros((_EPAD - _E,), jnp.float32)])
    zeros = jnp.zeros((_RPT, _HALF), jnp.float32)

    y0, st0 = _dense0(embed, W0, b0.reshape(1, _D))
    H0 = _norm(y0, st0, g0.reshape(1, _D), bt0.reshape(1, _D))

    S1 = _propagate(H0.reshape(2 * _N, _HALF), col, row, val, zeros)
    y1, st1 = _dense_halves(S1, W1, b1.reshape(1, _D))
    H1 = _norm(y1, st1, g1.reshape(1, _D), bt1.reshape(1, _D))

    S2 = _propagate(H1.reshape(2 * _N, _HALF), col, row, val, zeros)
    y2, st2 = _dense_halves(S2, W2, b2.reshape(1, _D))
    H2 = _norm(y2, st2, g2.reshape(1, _D), bt2.reshape(1, _D))

    return (H0, H1, H2)
